# baseline (speedup 1.0000x reference)
import functools

import jax
import jax.numpy as jnp
from jax import lax
from jax.experimental import pallas as pl
from jax.experimental.pallas import tpu as pltpu
from jax.experimental.pallas import tpu_sc as plsc

NC = 2
NS = 16
L = 16
NW = NC * NS
TOTAL = 16384 * 100
PER_W = TOTAL // NW
C = 8
P = 4
PIECE = PER_W // P
U = 8
PVECS = PIECE // L


def _sc_body(x_hbm, w_hbm, out_hbm, x_v, w_v, acc_v, hist_v, sem0, sem1):
    cid = lax.axis_index("c")
    sid = lax.axis_index("s")
    wid = cid * NS + sid
    base = wid * PER_W
    sems = [sem0, sem1]

    copies = [None] * P
    for g in range(2):
        copies[g] = pltpu.async_copy(
            x_hbm.at[pl.ds(base + g * PIECE, PIECE)], x_v.at[g], sems[g])

    pltpu.sync_copy(w_hbm, w_v)
    s = w_v[0, :]
    for c in range(1, 2 * L):
        s = s + w_v[c, :]

    zero = jnp.zeros((L,), jnp.float32)
    for r in range(C * L):
        hist_v[pl.ds(r * L, L)] = zero

    ones = jnp.full((L,), 1.0, jnp.float32)
    lane = lax.iota(jnp.int32, L)
    offs = [lane + jnp.int32(c * L * L) for c in range(C)]

    for g in range(P):
        copies[g].wait()
        buf = g % 2

        def body(i, _, buf=buf):
            for j in range(U):
                idx = x_v[buf, pl.ds(i * (U * L) + j * L, L)]
                vidx = (idx << 4) + offs[j % C]
                plsc.addupdate_scatter(hist_v, [vidx], ones)
            return 0

        lax.fori_loop(0, PVECS // U, body, 0)
        if g + 2 < P:
            copies[g + 2] = pltpu.async_copy(
                x_hbm.at[pl.ds(base + (g + 2) * PIECE, PIECE)],
                x_v.at[buf], sems[buf])

    acc = jnp.zeros((L,), jnp.float32)
    for c in range(C):
        for r in range(10):
            row = hist_v[pl.ds(c * L * L + r * L, L)]
            acc = acc + row * s[r]
    acc_v[...] = acc
    pltpu.sync_copy(acc_v, out_hbm.at[wid])


_sc_call = functools.partial(
    pl.kernel,
    out_type=jax.ShapeDtypeStruct((NW, L), jnp.float32),
    mesh=plsc.VectorSubcoreMesh(core_axis_name="c", subcore_axis_name="s"),
    compiler_params=pltpu.CompilerParams(needs_layout_passes=False),
    scratch_types=[
        pltpu.VMEM((2, PIECE), jnp.int32),
        pltpu.VMEM((2 * L, L), jnp.float32),
        pltpu.VMEM((L,), jnp.float32),
        pltpu.VMEM((C * L * L,), jnp.float32),
        pltpu.SemaphoreType.DMA,
        pltpu.SemaphoreType.DMA,
    ],
)


def kernel(x, weight):
    x_flat = x.reshape(-1).astype(jnp.int32)
    w_t = jnp.zeros((2 * L, L), jnp.float32).at[:20, :10].set(
        weight.astype(jnp.float32).T)
    out = _sc_call(_sc_body)(x_flat, w_t)
    return out.sum()


# speedup vs baseline: 1.3600x; 1.3600x over previous
import functools

import jax
import jax.numpy as jnp
from jax import lax
from jax.experimental import pallas as pl
from jax.experimental.pallas import tpu as pltpu
from jax.experimental.pallas import tpu_sc as plsc

NC = 2
NS = 16
L = 16
NW = NC * NS
TOTAL = 16384 * 100
PER_W = TOTAL // NW
C = 8
P = 4
PIECE = PER_W // P
U = 8
PVECS = PIECE // L
NROW = 16384
FEAT = 100
ROWS_W = NROW // NW
ROWS_P = ROWS_W // P


def _sc_body(x_hbm, w_hbm, out_hbm, x_v, w_v, acc_v, hist_v, sem0, sem1):
    cid = lax.axis_index("c")
    sid = lax.axis_index("s")
    wid = cid * NS + sid
    base = wid * PER_W
    sems = [sem0, sem1]

    copies = [None] * P
    for g in range(2):
        copies[g] = pltpu.async_copy(
            x_hbm.at[pl.ds(base + g * PIECE, PIECE)], x_v.at[g], sems[g])

    pltpu.sync_copy(w_hbm, w_v)
    s = w_v[0, :]
    for c in range(1, 2 * L):
        s = s + w_v[c, :]

    zero = jnp.zeros((L,), jnp.float32)
    for r in range(C * L):
        hist_v[pl.ds(r * L, L)] = zero

    ones = jnp.full((L,), 1.0, jnp.float32)
    lane = lax.iota(jnp.int32, L)

    for g in range(P):
        copies[g].wait()
        buf = g % 2

        @plsc.parallel_loop(0, PVECS, unroll=U)
        def _scatter(i, buf=buf):
            idx = x_v[buf, pl.ds(i * L, L)]
            coff = (i & (C - 1)) << 8
            vidx = ((idx << 4) | lane) + coff
            plsc.addupdate_scatter(hist_v, [vidx], ones)

        if g + 2 < P:
            copies[g + 2] = pltpu.async_copy(
                x_hbm.at[pl.ds(base + (g + 2) * PIECE, PIECE)],
                x_v.at[buf], sems[buf])

    acc = jnp.zeros((L,), jnp.float32)
    for c in range(C):
        for r in range(10):
            row = hist_v[pl.ds(c * L * L + r * L, L)]
            acc = acc + row * s[r]
    acc_v[...] = acc
    pltpu.sync_copy(acc_v, out_hbm.at[wid])


_sc_call = functools.partial(
    pl.kernel,
    out_type=jax.ShapeDtypeStruct((NW, L), jnp.float32),
    mesh=plsc.VectorSubcoreMesh(core_axis_name="c", subcore_axis_name="s"),
    compiler_params=pltpu.CompilerParams(needs_layout_passes=False),
    scratch_types=[
        pltpu.VMEM((2, PIECE), jnp.int32),
        pltpu.VMEM((2 * L, L), jnp.float32),
        pltpu.VMEM((L,), jnp.float32),
        pltpu.VMEM((C * L * L,), jnp.float32),
        pltpu.SemaphoreType.DMA,
        pltpu.SemaphoreType.DMA,
    ],
)


def kernel(x, weight):
    x_flat = x.reshape(-1).astype(jnp.int32)
    w_t = jnp.zeros((2 * L, L), jnp.float32).at[:20, :10].set(
        weight.astype(jnp.float32).T)
    out = _sc_call(_sc_body)(x_flat, w_t)
    return out.sum()


# speedup vs baseline: 2.3178x; 1.7043x over previous
import functools

import jax
import jax.numpy as jnp
from jax import lax
from jax.experimental import pallas as pl
from jax.experimental.pallas import tpu as pltpu
from jax.experimental.pallas import tpu_sc as plsc

NC = 2
NS = 16
L = 16
NW = NC * NS
NROW = 16384
FEAT = 100
NWIN = 8
C = 8
P = 4
COLS_W = NROW // NW
COLS_P = COLS_W // P
VROWS = 10
VCOLS = 20


def _sc_body(x_hbm, w_hbm, out_hbm, x_v0, x_v1, x_v2, x_v3, w_v, acc_v,
             hist_v, sem0, sem1, sem2, sem3):
    cid = lax.axis_index("c")
    sid = lax.axis_index("s")
    wid = cid * NS + sid
    cbase = wid * COLS_W
    sems = [sem0, sem1, sem2, sem3]
    bufs = [x_v0, x_v1, x_v2, x_v3]

    copies = [
        pltpu.async_copy(
            x_hbm.at[:, pl.ds(cbase + g * COLS_P, COLS_P)], bufs[g], sems[g])
        for g in range(P)
    ]
    pltpu.sync_copy(w_hbm, w_v)

    zero = jnp.zeros((L,), jnp.float32)
    for r in range(C * L):
        hist_v[pl.ds(r * L, L)] = zero

    ones = jnp.full((L,), 1.0, jnp.float32)
    lane = lax.iota(jnp.int32, L)
    woffs = [lane + jnp.int32((w % C) << 8) for w in range(NWIN)]

    for g in range(P):
        copies[g].wait()

        @plsc.parallel_loop(0, FEAT, unroll=2)
        def _scatter(r, buf=bufs[g]):
            for w in range(NWIN):
                idx = buf[r, pl.ds(w * L, L)]
                vidx = (idx << 4) | woffs[w]
                plsc.addupdate_scatter(hist_v, [vidx], ones)

    s = []
    for r in range(VROWS):
        v1 = w_v[r, pl.ds(0, L)]
        v2 = w_v[r, pl.ds(VCOLS - L, L)]
        tail = jnp.where(lane >= jnp.int32(2 * L - VCOLS), v2, 0.0)
        s.append(jnp.sum(v1) + jnp.sum(tail))

    acc = jnp.zeros((L,), jnp.float32)
    for c in range(C):
        for r in range(VROWS):
            row = hist_v[pl.ds(c * L * L + r * L, L)]
            acc = acc + row * s[r]
    acc_v[...] = acc
    pltpu.sync_copy(acc_v, out_hbm.at[wid])


_sc_call = functools.partial(
    pl.kernel,
    out_type=jax.ShapeDtypeStruct((NW, L), jnp.float32),
    mesh=plsc.VectorSubcoreMesh(core_axis_name="c", subcore_axis_name="s"),
    compiler_params=pltpu.CompilerParams(
        needs_layout_passes=False, use_tc_tiling_on_sc=True),
    scratch_types=[
        pltpu.VMEM((FEAT, COLS_P), jnp.int32),
        pltpu.VMEM((FEAT, COLS_P), jnp.int32),
        pltpu.VMEM((FEAT, COLS_P), jnp.int32),
        pltpu.VMEM((FEAT, COLS_P), jnp.int32),
        pltpu.VMEM((VROWS, VCOLS), jnp.float32),
        pltpu.VMEM((L,), jnp.float32),
        pltpu.VMEM((C * L * L,), jnp.float32),
        pltpu.SemaphoreType.DMA,
        pltpu.SemaphoreType.DMA,
        pltpu.SemaphoreType.DMA,
        pltpu.SemaphoreType.DMA,
    ],
)


def kernel(x, weight):
    out = _sc_call(_sc_body)(x.T, weight)
    return out.sum()


# speedup vs baseline: 2.3511x; 1.0144x over previous
import functools

import jax
import jax.numpy as jnp
from jax import lax
from jax.experimental import pallas as pl
from jax.experimental.pallas import tpu as pltpu
from jax.experimental.pallas import tpu_sc as plsc

NC = 2
NS = 16
L = 16
NW = NC * NS
NROW = 16384
FEAT = 100
NWIN = 8
C = 8
P = 4
COLS_W = NROW // NW
COLS_P = COLS_W // P
VROWS = 10
VCOLS = 20


def _sc_body(x_hbm, w_hbm, out_hbm, x_v0, x_v1, x_v2, x_v3, w_v, acc_v,
             hist_v, sem0, sem1, sem2, sem3):
    cid = lax.axis_index("c")
    sid = lax.axis_index("s")
    wid = cid * NS + sid
    cbase = wid * COLS_W
    sems = [sem0, sem1, sem2, sem3]
    bufs = [x_v0, x_v1, x_v2, x_v3]

    copies = [
        pltpu.async_copy(
            x_hbm.at[:, pl.ds(cbase + g * COLS_P, COLS_P)], bufs[g], sems[g])
        for g in range(P)
    ]
    pltpu.sync_copy(w_hbm, w_v)

    zero = jnp.zeros((L,), jnp.float32)
    for r in range(C * L):
        hist_v[pl.ds(r * L, L)] = zero

    ones = jnp.full((L,), 1.0, jnp.float32)
    lane = lax.iota(jnp.int32, L)
    woffs = [lane + jnp.int32((w % C) << 8) for w in range(NWIN)]

    for g in range(P):
        copies[g].wait()

        @plsc.parallel_loop(0, FEAT, unroll=2)
        def _scatter(r, buf=bufs[g]):
            for w in range(NWIN):
                idx = buf[r, pl.ds(w * L, L)]
                vidx = (idx << 4) | woffs[w]
                plsc.addupdate_scatter(hist_v, [vidx], ones)

    s = []
    for r in range(VROWS):
        v1 = w_v[r, pl.ds(0, L)]
        v2 = w_v[r, pl.ds(VCOLS - L, L)]
        tail = jnp.where(lane >= jnp.int32(2 * L - VCOLS), v2, 0.0)
        s.append(jnp.sum(v1) + jnp.sum(tail))

    accs = [jnp.zeros((L,), jnp.float32) for _ in range(4)]
    k = 0
    for c in range(C):
        for r in range(VROWS):
            row = hist_v[pl.ds(c * L * L + r * L, L)]
            accs[k % 4] = accs[k % 4] + row * s[r]
            k += 1
    acc = (accs[0] + accs[1]) + (accs[2] + accs[3])
    acc_v[...] = acc
    pltpu.sync_copy(acc_v, out_hbm.at[wid])


_sc_call = functools.partial(
    pl.kernel,
    out_type=jax.ShapeDtypeStruct((NW, L), jnp.float32),
    mesh=plsc.VectorSubcoreMesh(core_axis_name="c", subcore_axis_name="s"),
    compiler_params=pltpu.CompilerParams(
        needs_layout_passes=False, use_tc_tiling_on_sc=True),
    scratch_types=[
        pltpu.VMEM((FEAT, COLS_P), jnp.int32),
        pltpu.VMEM((FEAT, COLS_P), jnp.int32),
        pltpu.VMEM((FEAT, COLS_P), jnp.int32),
        pltpu.VMEM((FEAT, COLS_P), jnp.int32),
        pltpu.VMEM((VROWS, VCOLS), jnp.float32),
        pltpu.VMEM((L,), jnp.float32),
        pltpu.VMEM((C * L * L,), jnp.float32),
        pltpu.SemaphoreType.DMA,
        pltpu.SemaphoreType.DMA,
        pltpu.SemaphoreType.DMA,
        pltpu.SemaphoreType.DMA,
    ],
)


def kernel(x, weight):
    out = _sc_call(_sc_body)(x.T, weight)
    return out.sum()
